# baseline (speedup 1.0000x reference)
import functools

import jax
import jax.numpy as jnp
import numpy as np
from jax import lax
from jax.experimental import pallas as pl
from jax.experimental.pallas import tpu as pltpu
from jax.experimental.pallas import tpu_sc as plsc

N = 10000
E = 320000
H = 128
L = 3
NF = 2000
FE = 30000
NAF = 9
AV = 128
NBF = 3
BV = 8
INS = 20
B = 256
OUT = 1

NC = 2
NS = 16
NT = NC * NS

NPAD = 10240
TRASH = 10200
RPT = NPAD // NT

ECHUNK = 128
ECPT = 84
EPT = ECPT * ECHUNK
EPAD = NT * EPT

FCHUNK = 128
FCPT = 18
FEPAD = NS * FCPT * FCHUNK

GP = 384
GTRASH = 300

_f32 = jnp.float32
_i32 = jnp.int32

_mesh = plsc.VectorSubcoreMesh(core_axis_name="c", subcore_axis_name="s")


def _zero_rows(ref, nrows, ngroups):
    def body(r, _):
        for j in range(ngroups):
            ref[r, pl.ds(j * 16, 16)] = jnp.zeros((16,), _f32)
        return 0
    lax.fori_loop(0, nrows, body, 0)


def _fill_ones(ref, nrows, ngroups):
    def body(r, _):
        for j in range(ngroups):
            ref[r, pl.ds(j * 16, 16)] = jnp.ones((16,), _f32)
        return 0
    lax.fori_loop(0, nrows, body, 0)


@functools.partial(
    pl.kernel,
    out_type=jax.ShapeDtypeStruct((NPAD, H), _f32),
    mesh=_mesh,
    scratch_types=(
        pltpu.VMEM((576,), _i32),
        pltpu.VMEM((576, H), _f32),
        pltpu.VMEM((64, H), _f32),
        pltpu.SemaphoreType.DMA,
    ),
)
def _sc_atom(xoffp, atab, h0, idxv, abuf, hbuf, sem):
    c = lax.axis_index("c")
    s = lax.axis_index("s")
    t = c * NS + s

    def atom_chunk(k, _):
        pltpu.sync_copy(xoffp.at[t, k], idxv)
        cps = []
        for j in range(4):
            cps.append(pltpu.async_copy(
                atab.at[idxv.at[pl.ds(j * 128, 128)]],
                abuf.at[pl.ds(j * 128, 128)], sem))
        cps.append(pltpu.async_copy(
            atab.at[idxv.at[pl.ds(512, 64)]], abuf.at[pl.ds(512, 64)], sem))
        for cp in cps:
            cp.wait()

        def srow(r, _):
            for j in range(8):
                acc = abuf[r, pl.ds(j * 16, 16)]
                for f in range(1, NAF):
                    acc = acc + abuf[f * 64 + r, pl.ds(j * 16, 16)]
                hbuf[r, pl.ds(j * 16, 16)] = acc
            return 0
        lax.fori_loop(0, 64, srow, 0)
        base = pl.multiple_of(t * RPT + k * 64, 64)
        pltpu.sync_copy(hbuf, h0.at[pl.ds(base, 64)])
        return 0
    lax.fori_loop(0, 5, atom_chunk, 0)


@functools.partial(
    pl.kernel,
    out_type=jax.ShapeDtypeStruct((NPAD, H), _f32),
    mesh=_mesh,
    scratch_types=(
        pltpu.VMEM((2, FCHUNK), _i32),
        pltpu.VMEM((FCHUNK, H), _f32),
        pltpu.VMEM((64, H), _f32),
        pltpu.VMEM_SHARED((NPAD, H), _f32),
        pltpu.SemaphoreType.DMA,
    ),
)
def _sc_frag(fpack, xfrag, fsum, fidx, fbuf, hbuf, ssum_sp, sem):
    c = lax.axis_index("c")
    s = lax.axis_index("s")
    r0 = s * 640

    @pl.when(c == 0)
    def _():
        _zero_rows(hbuf, 64, 8)

        def zslice(k, _):
            pltpu.sync_copy(hbuf, ssum_sp.at[pl.ds(r0 + k * 64, 64)])
            return 0
        lax.fori_loop(0, 10, zslice, 0)

    plsc.subcore_barrier()

    @pl.when(c == 0)
    def _():
        def frag_chunk(k, _):
            g = s * FCPT + k
            pltpu.sync_copy(fpack.at[g], fidx)
            pltpu.async_copy(xfrag.at[fidx.at[0]], fbuf, sem).wait()
            pltpu.sync_copy(fbuf, ssum_sp.at[fidx.at[1]], add=True)
            return 0
        lax.fori_loop(0, FCPT, frag_chunk, 0)

    plsc.subcore_barrier()

    @pl.when(c == 0)
    def _():
        def dump(b, _):
            pltpu.sync_copy(ssum_sp.at[pl.ds(r0 + b * 64, 64)], hbuf)
            pltpu.sync_copy(hbuf, fsum.at[pl.ds(r0 + b * 64, 64)])
            return 0
        lax.fori_loop(0, 10, dump, 0)


@functools.partial(
    pl.kernel,
    out_type=jax.ShapeDtypeStruct((NPAD, H), _f32),
    mesh=_mesh,
    scratch_types=(
        pltpu.VMEM((2, FCHUNK), _i32),
        pltpu.VMEM((FCHUNK, H), _f32),
        pltpu.VMEM((64, H), _f32),
        pltpu.VMEM_SHARED((NPAD, H), _f32),
    ),
)
def _sc_fcnt(fpack, cnt, fidx, ones, hbuf, cnt_sp):
    c = lax.axis_index("c")
    s = lax.axis_index("s")
    r0 = s * 640

    @pl.when(c == 0)
    def _():
        _zero_rows(hbuf, 64, 8)
        _fill_ones(ones, FCHUNK, 8)

        def zslice(k, _):
            pltpu.sync_copy(hbuf, cnt_sp.at[pl.ds(r0 + k * 64, 64)])
            return 0
        lax.fori_loop(0, 10, zslice, 0)

    plsc.subcore_barrier()

    @pl.when(c == 0)
    def _():
        def frag_chunk(k, _):
            g = s * FCPT + k
            pltpu.sync_copy(fpack.at[g], fidx)
            pltpu.sync_copy(ones, cnt_sp.at[fidx.at[1]], add=True)
            return 0
        lax.fori_loop(0, FCPT, frag_chunk, 0)

    plsc.subcore_barrier()

    @pl.when(c == 0)
    def _():
        def dump(b, _):
            pltpu.sync_copy(cnt_sp.at[pl.ds(r0 + b * 64, 64)], hbuf)
            pltpu.sync_copy(hbuf, cnt.at[pl.ds(r0 + b * 64, 64)])
            return 0
        lax.fori_loop(0, 10, dump, 0)


@functools.partial(
    pl.kernel,
    out_type=jax.ShapeDtypeStruct((NC, NPAD, H), _f32),
    mesh=_mesh,
    scratch_types=(
        pltpu.VMEM((3, ECHUNK), _i32),
        pltpu.VMEM((ECHUNK, H), _f32),
        pltpu.VMEM((ECHUNK, H), _f32),
        pltpu.VMEM_SHARED((NPAD, H), _f32),
        pltpu.SemaphoreType.DMA,
        pltpu.SemaphoreType.DMA,
    ),
)
def _sc_edge(h, epack, tbl, agg, ebuf, hbuf, tbuf, agg_sp, sem1, sem2):
    c = lax.axis_index("c")
    s = lax.axis_index("s")
    t = c * NS + s

    _zero_rows(hbuf, 128, 8)
    r0 = s * 640
    for k in range(5):
        pltpu.sync_copy(hbuf, agg_sp.at[pl.ds(r0 + k * 128, 128)])
    plsc.subcore_barrier()

    def chunk(k, _):
        g = t * ECPT + k
        pltpu.sync_copy(epack.at[g], ebuf)
        cp1 = pltpu.async_copy(h.at[ebuf.at[0]], hbuf, sem1)
        cp2 = pltpu.async_copy(tbl.at[ebuf.at[2]], tbuf, sem2)
        cp1.wait()
        cp2.wait()

        def row(r, _):
            for j in range(8):
                a = hbuf[r, pl.ds(j * 16, 16)]
                b = tbuf[r, pl.ds(j * 16, 16)]
                hbuf[r, pl.ds(j * 16, 16)] = jnp.maximum(a + b, 0.0)
            return 0
        lax.fori_loop(0, ECHUNK, row, 0)
        pltpu.sync_copy(hbuf, agg_sp.at[ebuf.at[1]], add=True)
        return 0
    lax.fori_loop(0, ECPT, chunk, 0)
    plsc.subcore_barrier()

    for k in range(5):
        pltpu.sync_copy(agg_sp.at[pl.ds(r0 + k * 128, 128)], hbuf)
        pltpu.sync_copy(hbuf, agg.at[c, pl.ds(r0 + k * 128, 128)])


@functools.partial(
    pl.kernel,
    out_type=(
        jax.ShapeDtypeStruct((NC, GP, H), _f32),
        jax.ShapeDtypeStruct((NC, GP, H), _f32),
    ),
    mesh=_mesh,
    scratch_types=(
        pltpu.VMEM((64,), _i32),
        pltpu.VMEM((64, H), _f32),
        pltpu.VMEM((64, H), _f32),
        pltpu.VMEM((24, H), _f32),
        pltpu.VMEM_SHARED((GP, H), _f32),
        pltpu.VMEM_SHARED((GP, H), _f32),
    ),
)
def _sc_pool(h, bpack, gs, gc, bidx, hbuf, ones, czb, gs_sp, gc_sp):
    c = lax.axis_index("c")
    s = lax.axis_index("s")
    t = c * NS + s

    _zero_rows(hbuf, 64, 8)
    _fill_ones(ones, 64, 8)
    _zero_rows(czb, 24, 8)

    pltpu.sync_copy(hbuf.at[pl.ds(0, 24)], gs_sp.at[pl.ds(s * 24, 24)])
    pltpu.sync_copy(czb, gc_sp.at[pl.ds(s * 24, 24)])
    plsc.subcore_barrier()

    def chunk(k, _):
        base = pl.multiple_of(t * RPT + k * 64, 64)
        pltpu.sync_copy(h.at[pl.ds(base, 64)], hbuf)
        pltpu.sync_copy(bpack.at[t * 5 + k], bidx)
        pltpu.sync_copy(hbuf, gs_sp.at[bidx], add=True)
        pltpu.sync_copy(ones, gc_sp.at[bidx], add=True)
        return 0
    lax.fori_loop(0, 5, chunk, 0)
    plsc.subcore_barrier()

    pltpu.sync_copy(gs_sp.at[pl.ds(s * 24, 24)], hbuf.at[pl.ds(0, 24)])
    pltpu.sync_copy(hbuf.at[pl.ds(0, 24)], gs.at[c, pl.ds(s * 24, 24)])
    pltpu.sync_copy(gc_sp.at[pl.ds(s * 24, 24)], czb)
    pltpu.sync_copy(czb, gc.at[c, pl.ds(s * 24, 24)])


_S0 = np.eye(BV, dtype=np.float32)[
    (np.arange(BV * BV * BV) // (BV * BV)) % BV]
_S1 = np.eye(BV, dtype=np.float32)[(np.arange(BV * BV * BV) // BV) % BV]
_S2 = np.eye(BV, dtype=np.float32)[np.arange(BV * BV * BV) % BV]


def _tc_prep_body(frg, fw, fb, be, lw, lb, s0, s1, s2, xf_ref, t_ref):
    xf_ref[...] = jnp.dot(frg[...], fw[...],
                          preferred_element_type=_f32) + fb[...]
    for l in range(L):
        hi = jax.lax.Precision.HIGHEST
        b3 = (jnp.dot(s0[...], be[l, 0], preferred_element_type=_f32,
                      precision=hi)
              + jnp.dot(s1[...], be[l, 1], preferred_element_type=_f32,
                        precision=hi)
              + jnp.dot(s2[...], be[l, 2], preferred_element_type=_f32,
                        precision=hi))
        t_ref[l] = jnp.dot(b3, lw[l], preferred_element_type=_f32) + lb[l]


def _tc_prep(frg, fw, fb, be, lw, lb):
    return pl.pallas_call(
        _tc_prep_body,
        out_shape=(
            jax.ShapeDtypeStruct((NF, H), _f32),
            jax.ShapeDtypeStruct((L, BV * BV * BV, H), _f32),
        ),
    )(frg, fw, fb, be, lw, lb, _S0, _S1, _S2)


def _row_mask():
    return lax.broadcasted_iota(_i32, (NPAD, 1), 0) < N


def _tc_hinit_body(h0, fs, ct, out):
    mean = fs[...] / jnp.maximum(ct[...][:, 0:1], 1.0)
    out[...] = jnp.where(_row_mask(), h0[...] + mean, 0.0)


def _tc_hinit(h0, fsum, cnt):
    return pl.pallas_call(
        _tc_hinit_body,
        out_shape=jax.ShapeDtypeStruct((NPAD, H), _f32),
    )(h0, fsum, cnt)


def _bn_masked(z, g, b, mask):
    zm = jnp.where(mask, z, 0.0)
    mean = jnp.sum(zm, axis=0) / N
    d = jnp.where(mask, z - mean, 0.0)
    var = jnp.sum(d * d, axis=0) / N
    return (z - mean) / jnp.sqrt(var + 1e-5) * g + b


def _tc_layer_body(h, agg, ep, w1, b1, g1, be1, w2, b2, g2, be2, out):
    mask = _row_mask()
    z = h[...] * ep[0, 0] + (agg[0] + agg[1])
    z = jnp.where(mask, z, 0.0)
    z1 = jnp.dot(z, w1[...], preferred_element_type=_f32) + b1[...]
    y = jnp.maximum(_bn_masked(z1, g1[...], be1[...], mask), 0.0)
    z2 = jnp.dot(y, w2[...], preferred_element_type=_f32) + b2[...]
    out[...] = jnp.where(
        mask, jnp.maximum(_bn_masked(z2, g2[...], be2[...], mask), 0.0), 0.0)


def _tc_layer(h, agg, ep, w1, b1, g1, be1, w2, b2, g2, be2):
    return pl.pallas_call(
        _tc_layer_body,
        out_shape=jax.ShapeDtypeStruct((NPAD, H), _f32),
    )(h, agg, ep, w1, b1, g1, be1, w2, b2, g2, be2)


def _tc_out_body(gs, gc, aw1, ab1, aw2, ab2, ow1, ob1, ow2, ob2, out):
    num = (gs[0] + gs[1])[:B]
    den = jnp.maximum((gc[0] + gc[1])[:B, 0:1], 1.0)
    g = num / den
    g = jnp.maximum(jnp.dot(g, aw1[...], preferred_element_type=_f32)
                    + ab1[...], 0.0)
    g = jnp.maximum(jnp.dot(g, aw2[...], preferred_element_type=_f32)
                    + ab2[...], 0.0)
    o = jnp.maximum(jnp.dot(g, ow1[...], preferred_element_type=_f32)
                    + ob1[...], 0.0)
    out[...] = jnp.dot(o, ow2[...], preferred_element_type=_f32) + ob2[...]


def _tc_out(gs, gc, aw1, ab1, aw2, ab2, ow1, ob1, ow2, ob2):
    return pl.pallas_call(
        _tc_out_body,
        out_shape=jax.ShapeDtypeStruct((B, OUT), _f32),
    )(gs, gc, aw1, ab1, aw2, ab2, ow1, ob1, ow2, ob2)


def kernel(x, edge_index, edge_attr, batch, frag_row, frag_col, fragments,
           atom_emb, frag_W, frag_b, bond_emb, eps, lin_edge_W, lin_edge_b,
           nn_W1, nn_b1, nn_g1, nn_be1, nn_W2, nn_b2, bn_g, bn_be,
           aout_W1, aout_b1, aout_W2, aout_b2, out_W1, out_b1, out_W2,
           out_b2):
    x = x.astype(_i32)
    xoff = x.T + (jnp.arange(NAF, dtype=_i32) * AV)[:, None]
    xoff = jnp.pad(xoff, ((0, 0), (0, NPAD - N)))
    xoffp = xoff.reshape(NAF, NT, 5, 64).transpose(1, 2, 0, 3) \
                .reshape(NT, 5, NAF * 64)
    atab = atom_emb.reshape(NAF * AV, H)

    def _slot_pack(vals, owner, key_fill, ntiles, cap, nslots):
        ne = owner.shape[0]
        order = jnp.argsort(owner, stable=True)
        sowner = owner[order]
        counts = jnp.bincount(sowner, length=ntiles)
        offs = jnp.concatenate(
            [jnp.zeros((1,), _i32),
             jnp.cumsum(counts).astype(_i32)[:-1]])
        pos = jnp.arange(ne, dtype=_i32) - offs[sowner]
        slots = sowner * cap + pos
        fallback = counts.max() > cap
        packed = []
        for v, fill in zip(vals, key_fill):
            vo = jnp.full((nslots,), fill, _i32).at[slots].set(v[order])
            vp = jnp.full((nslots,), fill, _i32).at[
                jnp.arange(ne, dtype=_i32)].set(v)
            packed.append(jnp.where(fallback, vp, vo))
        return packed

    fcol = frag_col.astype(_i32)
    frow = frag_row.astype(_i32)
    fc_p, fr_p = _slot_pack([fcol, frow], frow // 640, [0, TRASH],
                            NS, FCPT * FCHUNK, FEPAD)
    fpack = jnp.stack([fc_p.reshape(-1, FCHUNK), fr_p.reshape(-1, FCHUNK)],
                      axis=1)

    ea = edge_attr.astype(_i32)
    codes = ea[:, 0] * (BV * BV) + ea[:, 1] * BV + ea[:, 2]
    src = edge_index[0].astype(_i32)
    dst = edge_index[1].astype(_i32)
    s_p, d_p, c_p = _slot_pack([src, dst, codes], dst // RPT,
                               [0, TRASH, 0], NT, EPT, EPAD)
    epack = jnp.stack([s_p.reshape(-1, ECHUNK), d_p.reshape(-1, ECHUNK),
                       c_p.reshape(-1, ECHUNK)], axis=1)

    bpack = jnp.pad(batch.astype(_i32), (0, NPAD - N),
                    constant_values=GTRASH).reshape(NT * 5, 64)

    fragments_p = jnp.pad(fragments, ((0, 0), (0, 32 - INS)))
    frag_Wp = jnp.pad(frag_W, ((0, 32 - INS), (0, 0)))

    xfrag, tbl = _tc_prep(fragments_p, frag_Wp, frag_b, bond_emb,
                          lin_edge_W, lin_edge_b)
    h0 = _sc_atom(xoffp, atab)
    fsum = _sc_frag(fpack, xfrag)
    fcnt = _sc_fcnt(fpack)
    h = _tc_hinit(h0, fsum, fcnt)
    for i in range(L):
        agg = _sc_edge(h, epack, tbl[i])
        h = _tc_layer(h, agg, (1.0 + eps[i]).reshape(1, 1),
                      nn_W1[i], nn_b1[i], nn_g1[i], nn_be1[i],
                      nn_W2[i], nn_b2[i], bn_g[i], bn_be[i])
    gs, gc = _sc_pool(h, bpack)
    return _tc_out(gs, gc, aout_W1, aout_b1, aout_W2, aout_b2,
                   out_W1, out_b1, out_W2, out_b2)


# speedup vs baseline: 1.5410x; 1.5410x over previous
import functools

import jax
import jax.numpy as jnp
import numpy as np
from jax import lax
from jax.experimental import pallas as pl
from jax.experimental.pallas import tpu as pltpu
from jax.experimental.pallas import tpu_sc as plsc

N = 10000
E = 320000
H = 128
L = 3
NF = 2000
FE = 30000
NAF = 9
AV = 128
NBF = 3
BV = 8
INS = 20
B = 256
OUT = 1

NC = 2
NS = 16
NT = NC * NS

NPAD = 10240
TRASH = 10200
RPT = NPAD // NT

ECHUNK = 128
ECPT = 84
EPT = ECPT * ECHUNK
EPAD = NT * EPT

FCHUNK = 128
FCPT = 18
FEPAD = NS * FCPT * FCHUNK

GP = 384
GTRASH = 300

_f32 = jnp.float32
_i32 = jnp.int32

_mesh = plsc.VectorSubcoreMesh(core_axis_name="c", subcore_axis_name="s")


def _zero_rows(ref, nrows, ngroups):
    def body(r, _):
        for j in range(ngroups):
            ref[r, pl.ds(j * 16, 16)] = jnp.zeros((16,), _f32)
        return 0
    lax.fori_loop(0, nrows, body, 0)


def _fill_ones(ref, nrows, ngroups):
    def body(r, _):
        for j in range(ngroups):
            ref[r, pl.ds(j * 16, 16)] = jnp.ones((16,), _f32)
        return 0
    lax.fori_loop(0, nrows, body, 0)


@functools.partial(
    pl.kernel,
    out_type=jax.ShapeDtypeStruct((NPAD, H), _f32),
    mesh=_mesh,
    scratch_types=(
        pltpu.VMEM((576,), _i32),
        pltpu.VMEM((576, H), _f32),
        pltpu.VMEM((64, H), _f32),
        pltpu.SemaphoreType.DMA,
    ),
)
def _sc_atom(xoffp, atab, h0, idxv, abuf, hbuf, sem):
    c = lax.axis_index("c")
    s = lax.axis_index("s")
    t = c * NS + s

    def atom_chunk(k, _):
        pltpu.sync_copy(xoffp.at[t, k], idxv)
        cps = []
        for j in range(4):
            cps.append(pltpu.async_copy(
                atab.at[idxv.at[pl.ds(j * 128, 128)]],
                abuf.at[pl.ds(j * 128, 128)], sem))
        cps.append(pltpu.async_copy(
            atab.at[idxv.at[pl.ds(512, 64)]], abuf.at[pl.ds(512, 64)], sem))
        for cp in cps:
            cp.wait()

        def srow(r, _):
            for j in range(8):
                acc = abuf[r, pl.ds(j * 16, 16)]
                for f in range(1, NAF):
                    acc = acc + abuf[f * 64 + r, pl.ds(j * 16, 16)]
                hbuf[r, pl.ds(j * 16, 16)] = acc
            return 0
        lax.fori_loop(0, 64, srow, 0)
        base = pl.multiple_of(t * RPT + k * 64, 64)
        pltpu.sync_copy(hbuf, h0.at[pl.ds(base, 64)])
        return 0
    lax.fori_loop(0, 5, atom_chunk, 0)


@functools.partial(
    pl.kernel,
    out_type=jax.ShapeDtypeStruct((NPAD, H), _f32),
    mesh=_mesh,
    scratch_types=(
        pltpu.VMEM((2, FCHUNK), _i32),
        pltpu.VMEM((FCHUNK, H), _f32),
        pltpu.VMEM((64, H), _f32),
        pltpu.VMEM_SHARED((NPAD, H), _f32),
        pltpu.SemaphoreType.DMA,
    ),
)
def _sc_frag(fpack, xfrag, fsum, fidx, fbuf, hbuf, ssum_sp, sem):
    c = lax.axis_index("c")
    s = lax.axis_index("s")
    r0 = s * 640

    @pl.when(c == 0)
    def _():
        _zero_rows(hbuf, 64, 8)

        def zslice(k, _):
            pltpu.sync_copy(hbuf, ssum_sp.at[pl.ds(r0 + k * 64, 64)])
            return 0
        lax.fori_loop(0, 10, zslice, 0)

    plsc.subcore_barrier()

    @pl.when(c == 0)
    def _():
        def frag_chunk(k, _):
            g = s * FCPT + k
            pltpu.sync_copy(fpack.at[g], fidx)
            pltpu.async_copy(xfrag.at[fidx.at[0]], fbuf, sem).wait()
            pltpu.sync_copy(fbuf, ssum_sp.at[fidx.at[1]], add=True)
            return 0
        lax.fori_loop(0, FCPT, frag_chunk, 0)

    plsc.subcore_barrier()

    @pl.when(c == 0)
    def _():
        def dump(b, _):
            pltpu.sync_copy(ssum_sp.at[pl.ds(r0 + b * 64, 64)], hbuf)
            pltpu.sync_copy(hbuf, fsum.at[pl.ds(r0 + b * 64, 64)])
            return 0
        lax.fori_loop(0, 10, dump, 0)


@functools.partial(
    pl.kernel,
    out_type=jax.ShapeDtypeStruct((NPAD, H), _f32),
    mesh=_mesh,
    scratch_types=(
        pltpu.VMEM((2, FCHUNK), _i32),
        pltpu.VMEM((FCHUNK, H), _f32),
        pltpu.VMEM((64, H), _f32),
        pltpu.VMEM_SHARED((NPAD, H), _f32),
    ),
)
def _sc_fcnt(fpack, cnt, fidx, ones, hbuf, cnt_sp):
    c = lax.axis_index("c")
    s = lax.axis_index("s")
    r0 = s * 640

    @pl.when(c == 0)
    def _():
        _zero_rows(hbuf, 64, 8)
        _fill_ones(ones, FCHUNK, 8)

        def zslice(k, _):
            pltpu.sync_copy(hbuf, cnt_sp.at[pl.ds(r0 + k * 64, 64)])
            return 0
        lax.fori_loop(0, 10, zslice, 0)

    plsc.subcore_barrier()

    @pl.when(c == 0)
    def _():
        def frag_chunk(k, _):
            g = s * FCPT + k
            pltpu.sync_copy(fpack.at[g], fidx)
            pltpu.sync_copy(ones, cnt_sp.at[fidx.at[1]], add=True)
            return 0
        lax.fori_loop(0, FCPT, frag_chunk, 0)

    plsc.subcore_barrier()

    @pl.when(c == 0)
    def _():
        def dump(b, _):
            pltpu.sync_copy(cnt_sp.at[pl.ds(r0 + b * 64, 64)], hbuf)
            pltpu.sync_copy(hbuf, cnt.at[pl.ds(r0 + b * 64, 64)])
            return 0
        lax.fori_loop(0, 10, dump, 0)


@functools.partial(
    pl.kernel,
    out_type=jax.ShapeDtypeStruct((NC, NPAD, H), _f32),
    mesh=_mesh,
    scratch_types=(
        pltpu.VMEM((3, ECHUNK), _i32),
        pltpu.VMEM((ECHUNK, H), _f32),
        pltpu.VMEM((ECHUNK, H), _f32),
        pltpu.VMEM_SHARED((NPAD, H), _f32),
        pltpu.SemaphoreType.DMA,
        pltpu.SemaphoreType.DMA,
    ),
)
def _sc_edge(h, epack, tbl, agg, ebuf, hbuf, tbuf, agg_sp, sem1, sem2):
    c = lax.axis_index("c")
    s = lax.axis_index("s")
    t = c * NS + s

    _zero_rows(hbuf, 128, 8)
    r0 = s * 640
    for k in range(5):
        pltpu.sync_copy(hbuf, agg_sp.at[pl.ds(r0 + k * 128, 128)])
    plsc.subcore_barrier()

    def chunk(k, _):
        g = t * ECPT + k
        pltpu.sync_copy(epack.at[g], ebuf)
        cp1 = pltpu.async_copy(h.at[ebuf.at[0]], hbuf, sem1)
        cp2 = pltpu.async_copy(tbl.at[ebuf.at[2]], tbuf, sem2)
        cp1.wait()
        cp2.wait()

        def row(r, _):
            for j in range(8):
                a = hbuf[r, pl.ds(j * 16, 16)]
                b = tbuf[r, pl.ds(j * 16, 16)]
                hbuf[r, pl.ds(j * 16, 16)] = jnp.maximum(a + b, 0.0)
            return 0
        lax.fori_loop(0, ECHUNK, row, 0)
        pltpu.sync_copy(hbuf, agg_sp.at[ebuf.at[1]], add=True)
        return 0
    lax.fori_loop(0, ECPT, chunk, 0)
    plsc.subcore_barrier()

    for k in range(5):
        pltpu.sync_copy(agg_sp.at[pl.ds(r0 + k * 128, 128)], hbuf)
        pltpu.sync_copy(hbuf, agg.at[c, pl.ds(r0 + k * 128, 128)])


@functools.partial(
    pl.kernel,
    out_type=(
        jax.ShapeDtypeStruct((NC, GP, H), _f32),
        jax.ShapeDtypeStruct((NC, GP, H), _f32),
    ),
    mesh=_mesh,
    scratch_types=(
        pltpu.VMEM((64,), _i32),
        pltpu.VMEM((64, H), _f32),
        pltpu.VMEM((64, H), _f32),
        pltpu.VMEM((24, H), _f32),
        pltpu.VMEM_SHARED((GP, H), _f32),
        pltpu.VMEM_SHARED((GP, H), _f32),
    ),
)
def _sc_pool(h, bpack, gs, gc, bidx, hbuf, ones, czb, gs_sp, gc_sp):
    c = lax.axis_index("c")
    s = lax.axis_index("s")
    t = c * NS + s

    _zero_rows(hbuf, 64, 8)
    _fill_ones(ones, 64, 8)
    _zero_rows(czb, 24, 8)

    pltpu.sync_copy(hbuf.at[pl.ds(0, 24)], gs_sp.at[pl.ds(s * 24, 24)])
    pltpu.sync_copy(czb, gc_sp.at[pl.ds(s * 24, 24)])
    plsc.subcore_barrier()

    def chunk(k, _):
        base = pl.multiple_of(t * RPT + k * 64, 64)
        pltpu.sync_copy(h.at[pl.ds(base, 64)], hbuf)
        pltpu.sync_copy(bpack.at[t * 5 + k], bidx)
        pltpu.sync_copy(hbuf, gs_sp.at[bidx], add=True)
        pltpu.sync_copy(ones, gc_sp.at[bidx], add=True)
        return 0
    lax.fori_loop(0, 5, chunk, 0)
    plsc.subcore_barrier()

    pltpu.sync_copy(gs_sp.at[pl.ds(s * 24, 24)], hbuf.at[pl.ds(0, 24)])
    pltpu.sync_copy(hbuf.at[pl.ds(0, 24)], gs.at[c, pl.ds(s * 24, 24)])
    pltpu.sync_copy(gc_sp.at[pl.ds(s * 24, 24)], czb)
    pltpu.sync_copy(czb, gc.at[c, pl.ds(s * 24, 24)])


_S0 = np.eye(BV, dtype=np.float32)[
    (np.arange(BV * BV * BV) // (BV * BV)) % BV]
_S1 = np.eye(BV, dtype=np.float32)[(np.arange(BV * BV * BV) // BV) % BV]
_S2 = np.eye(BV, dtype=np.float32)[np.arange(BV * BV * BV) % BV]


def _tc_prep_body(frg, fw, fb, be, lw, lb, s0, s1, s2, xf_ref, t_ref):
    xf_ref[...] = jnp.dot(frg[...], fw[...],
                          preferred_element_type=_f32) + fb[...]
    for l in range(L):
        hi = jax.lax.Precision.HIGHEST
        b3 = (jnp.dot(s0[...], be[l, 0], preferred_element_type=_f32,
                      precision=hi)
              + jnp.dot(s1[...], be[l, 1], preferred_element_type=_f32,
                        precision=hi)
              + jnp.dot(s2[...], be[l, 2], preferred_element_type=_f32,
                        precision=hi))
        t_ref[l] = jnp.dot(b3, lw[l], preferred_element_type=_f32) + lb[l]


def _tc_prep(frg, fw, fb, be, lw, lb):
    return pl.pallas_call(
        _tc_prep_body,
        out_shape=(
            jax.ShapeDtypeStruct((NF, H), _f32),
            jax.ShapeDtypeStruct((L, BV * BV * BV, H), _f32),
        ),
    )(frg, fw, fb, be, lw, lb, _S0, _S1, _S2)


def _row_mask():
    return lax.broadcasted_iota(_i32, (NPAD, 1), 0) < N


def _tc_hinit_body(h0, fs, ct, out):
    mean = fs[...] / jnp.maximum(ct[...][:, 0:1], 1.0)
    out[...] = jnp.where(_row_mask(), h0[...] + mean, 0.0)


def _tc_hinit(h0, fsum, cnt):
    return pl.pallas_call(
        _tc_hinit_body,
        out_shape=jax.ShapeDtypeStruct((NPAD, H), _f32),
    )(h0, fsum, cnt)


def _bn_masked(z, g, b, mask):
    zm = jnp.where(mask, z, 0.0)
    mean = jnp.sum(zm, axis=0) / N
    d = jnp.where(mask, z - mean, 0.0)
    var = jnp.sum(d * d, axis=0) / N
    return (z - mean) / jnp.sqrt(var + 1e-5) * g + b


def _tc_layer_body(h, agg, ep, w1, b1, g1, be1, w2, b2, g2, be2, out):
    mask = _row_mask()
    z = h[...] * ep[0, 0] + (agg[0] + agg[1])
    z = jnp.where(mask, z, 0.0)
    z1 = jnp.dot(z, w1[...], preferred_element_type=_f32) + b1[...]
    y = jnp.maximum(_bn_masked(z1, g1[...], be1[...], mask), 0.0)
    z2 = jnp.dot(y, w2[...], preferred_element_type=_f32) + b2[...]
    out[...] = jnp.where(
        mask, jnp.maximum(_bn_masked(z2, g2[...], be2[...], mask), 0.0), 0.0)


def _tc_layer(h, agg, ep, w1, b1, g1, be1, w2, b2, g2, be2):
    return pl.pallas_call(
        _tc_layer_body,
        out_shape=jax.ShapeDtypeStruct((NPAD, H), _f32),
    )(h, agg, ep, w1, b1, g1, be1, w2, b2, g2, be2)


def _tc_out_body(gs, gc, aw1, ab1, aw2, ab2, ow1, ob1, ow2, ob2, out):
    num = (gs[0] + gs[1])[:B]
    den = jnp.maximum((gc[0] + gc[1])[:B, 0:1], 1.0)
    g = num / den
    g = jnp.maximum(jnp.dot(g, aw1[...], preferred_element_type=_f32)
                    + ab1[...], 0.0)
    g = jnp.maximum(jnp.dot(g, aw2[...], preferred_element_type=_f32)
                    + ab2[...], 0.0)
    o = jnp.maximum(jnp.dot(g, ow1[...], preferred_element_type=_f32)
                    + ob1[...], 0.0)
    out[...] = jnp.dot(o, ow2[...], preferred_element_type=_f32) + ob2[...]


def _tc_out(gs, gc, aw1, ab1, aw2, ab2, ow1, ob1, ow2, ob2):
    return pl.pallas_call(
        _tc_out_body,
        out_shape=jax.ShapeDtypeStruct((B, OUT), _f32),
    )(gs, gc, aw1, ab1, aw2, ab2, ow1, ob1, ow2, ob2)


def kernel(x, edge_index, edge_attr, batch, frag_row, frag_col, fragments,
           atom_emb, frag_W, frag_b, bond_emb, eps, lin_edge_W, lin_edge_b,
           nn_W1, nn_b1, nn_g1, nn_be1, nn_W2, nn_b2, bn_g, bn_be,
           aout_W1, aout_b1, aout_W2, aout_b2, out_W1, out_b1, out_W2,
           out_b2):
    x = x.astype(_i32)
    xoff = x.T + (jnp.arange(NAF, dtype=_i32) * AV)[:, None]
    xoff = jnp.pad(xoff, ((0, 0), (0, NPAD - N)))
    xoffp = xoff.reshape(NAF, NT, 5, 64).transpose(1, 2, 0, 3) \
                .reshape(NT, 5, NAF * 64)
    atab = atom_emb.reshape(NAF * AV, H)

    def _slot_pack(vals, owner, key_fill, ntiles, cap, nslots):
        ne = owner.shape[0]
        oh = (owner[:, None] == jnp.arange(ntiles, dtype=_i32)[None, :]) \
            .astype(_i32)
        run = jnp.cumsum(oh, axis=0)
        counts = run[-1]
        pos = jnp.take_along_axis(run, owner[:, None], axis=1)[:, 0] - 1
        slots = owner * cap + pos
        fallback = counts.max() > cap
        slots = jnp.where(fallback, jnp.arange(ne, dtype=_i32), slots)
        packed = []
        for v, fill in zip(vals, key_fill):
            packed.append(jnp.full((nslots,), fill, _i32).at[slots].set(v))
        return packed

    fcol = frag_col.astype(_i32)
    frow = frag_row.astype(_i32)
    fc_p, fr_p = _slot_pack([fcol, frow], frow // 640, [0, TRASH],
                            NS, FCPT * FCHUNK, FEPAD)
    fpack = jnp.stack([fc_p.reshape(-1, FCHUNK), fr_p.reshape(-1, FCHUNK)],
                      axis=1)

    ea = edge_attr.astype(_i32)
    codes = ea[:, 0] * (BV * BV) + ea[:, 1] * BV + ea[:, 2]
    src = edge_index[0].astype(_i32)
    dst = edge_index[1].astype(_i32)
    s_p, d_p, c_p = _slot_pack([src, dst, codes], dst // RPT,
                               [0, TRASH, 0], NT, EPT, EPAD)
    epack = jnp.stack([s_p.reshape(-1, ECHUNK), d_p.reshape(-1, ECHUNK),
                       c_p.reshape(-1, ECHUNK)], axis=1)

    bpack = jnp.pad(batch.astype(_i32), (0, NPAD - N),
                    constant_values=GTRASH).reshape(NT * 5, 64)

    fragments_p = jnp.pad(fragments, ((0, 0), (0, 32 - INS)))
    frag_Wp = jnp.pad(frag_W, ((0, 32 - INS), (0, 0)))

    xfrag, tbl = _tc_prep(fragments_p, frag_Wp, frag_b, bond_emb,
                          lin_edge_W, lin_edge_b)
    h0 = _sc_atom(xoffp, atab)
    fsum = _sc_frag(fpack, xfrag)
    fcnt = _sc_fcnt(fpack)
    h = _tc_hinit(h0, fsum, fcnt)
    for i in range(L):
        agg = _sc_edge(h, epack, tbl[i])
        h = _tc_layer(h, agg, (1.0 + eps[i]).reshape(1, 1),
                      nn_W1[i], nn_b1[i], nn_g1[i], nn_be1[i],
                      nn_W2[i], nn_b2[i], bn_g[i], bn_be[i])
    gs, gc = _sc_pool(h, bpack)
    return _tc_out(gs, gc, aout_W1, aout_b1, aout_W2, aout_b2,
                   out_W1, out_b1, out_W2, out_b2)
